# Initial kernel scaffold; baseline (speedup 1.0000x reference)
import jax
import jax.numpy as jnp
from jax.experimental import pallas as pl


def kernel(p_v, y, y_pred):
    raise NotImplementedError("write your pallas kernel here")


# speedup vs baseline: 1.1882x; 1.1882x over previous
import functools

import jax
import jax.numpy as jnp
from jax.experimental import pallas as pl
from jax.experimental.pallas import tpu as pltpu

_N = 16384
_D = 4096
_BLK = 1024
_GRID = _N // _BLK


def _loss_body(p_ref, y_ref, yp_ref, out_ref, acc_ref, cnt_ref):
    j = pl.program_id(0)

    @pl.when(j == 0)
    def _init():
        acc_ref[...] = jnp.zeros_like(acc_ref)
        cnt_ref[0] = 0.0
        cnt_ref[1] = 0.0

    blk = p_ref[...]
    y = y_ref[...]
    yp = yp_ref[...]
    w_pos = (y >= 0).astype(jnp.float32)
    w_pp = (yp >= 0).astype(jnp.float32)
    ones = jnp.ones_like(w_pos)
    W = jnp.stack([ones, w_pos, w_pp], axis=0)
    acc_ref[0:3, :] += jnp.dot(W, blk, preferred_element_type=jnp.float32)
    cnt_ref[0] += jnp.sum(w_pos)
    cnt_ref[1] += jnp.sum(w_pp)

    @pl.when(j == _GRID - 1)
    def _finish():
        s_all = acc_ref[0, :]
        s_pos = acc_ref[1, :]
        s_pp = acc_ref[2, :]
        n = jnp.float32(_N)
        n_pos = cnt_ref[0]
        n_pp = cnt_ref[1]
        n_neg = n - n_pos

        pos_avg = s_pos / n_pos
        neg_avg = (s_all - s_pos) / n_neg
        pos_avg_p = s_pp / n_pp
        neg_avg_p = (s_all - s_pp) / (n - n_pp)

        def one_minus_cos(a, b):
            dot = jnp.sum(a * b)
            na = jnp.sqrt(jnp.sum(a * a))
            nb = jnp.sqrt(jnp.sum(b * b))
            return 1.0 - dot / jnp.maximum(na * nb, 1e-8)

        cp = one_minus_cos(pos_avg, pos_avg_p)
        cn = one_minus_cos(neg_avg, neg_avg_p)
        out_ref[0] = n_pos * cp / n + n_neg * cn / n


@jax.jit
def kernel(p_v, y, y_pred):
    out = pl.pallas_call(
        _loss_body,
        grid=(_GRID,),
        in_specs=[
            pl.BlockSpec((_BLK, _D), lambda j: (j, 0)),
            pl.BlockSpec((_BLK,), lambda j: (j,)),
            pl.BlockSpec((_BLK,), lambda j: (j,)),
        ],
        out_specs=pl.BlockSpec(memory_space=pltpu.SMEM),
        out_shape=jax.ShapeDtypeStruct((1,), jnp.float32),
        scratch_shapes=[
            pltpu.VMEM((8, _D), jnp.float32),
            pltpu.SMEM((2,), jnp.float32),
        ],
    )(p_v, y, y_pred)
    return out
